# baseline (speedup 1.0000x reference)
import dataclasses

import jax
import jax.numpy as jnp
from jax import lax
from jax.experimental import pallas as pl
from jax.experimental.pallas import tpu as pltpu
from jax.experimental.pallas import tpu_sc as plsc

BATCH = 4096
EMBED = 64
NROWS = 100000
ALIGNED = 99968
NR = 28
RW = 3584
WCH = 512
NPASS = 7
KMAX = 208
OUTROWS = BATCH + 8
LANE = 16

_GDN = lax.GatherDimensionNumbers(
    offset_dims=(), collapsed_slice_dims=(0,), start_index_map=(0,))


def _lane_bcast(vec, cj):
    return lax.gather(vec, cj[:, None], _GDN, slice_sizes=(1,),
                      mode=lax.GatherScatterMode.PROMISE_IN_BOUNDS)


def _iota16():
    return lax.iota(jnp.int32, 16)


def _spl(x):
    return jnp.full((LANE,), x, jnp.int32)


def _sweep_table(tab_hbm, tail_hbm, idx_hbm, out_hbm, idx_v, cmp_idx_v,
                 cmp_pos_v, ppos_v, chunk_v, tail_v, stag_v, sem, lo, hi, rng):
    for b in range(KMAX // LANE):
        ppos_v[pl.ds(LANE * b, LANE)] = _spl(BATCH)
        cmp_idx_v[pl.ds(LANE * b, LANE)] = _spl(-1)
        cmp_pos_v[pl.ds(LANE * b, LANE)] = _spl(BATCH)

    lo_spl = _spl(lo)
    hi_spl = _spl(hi)

    def sweep_body(s, slot_spl):
        pltpu.sync_copy(idx_hbm.at[pl.ds(1024 * s, 1024)], idx_v)

        def group_body(g, slot_spl):
            v = idx_v[pl.ds(LANE * g, LANE)]
            mask = (v >= lo_spl) & (v < hi_spl)
            pref = plsc.cumsum(mask.astype(jnp.int32))
            dst = slot_spl + pref - 1
            okm = mask & (dst < KMAX)
            pos_vec = _iota16() + (1024 * s + LANE * g)
            plsc.store_scatter(cmp_idx_v, [dst], v, mask=okm)
            plsc.store_scatter(cmp_pos_v, [dst], pos_vec, mask=okm)
            cnt_spl = plsc.all_reduce_population_count(mask)
            return slot_spl + cnt_spl

        return lax.fori_loop(0, 64, group_body, slot_spl)

    slot_spl = lax.fori_loop(0, 4, sweep_body, jnp.zeros((LANE,), jnp.int32))
    cnt = jnp.minimum(lax.reduce_max(slot_spl, axes=(0,)), KMAX)

    lane0 = _iota16() == 0

    def gather_pass(plo, phi, slot, from_tail):
        plo_spl = _spl(plo)
        phi_spl = _spl(phi)
        ngrp = (cnt + LANE - 1) // LANE

        def grp(gg, slot):
            m = cmp_idx_v[pl.ds(LANE * gg, LANE)]
            pv = cmp_pos_v[pl.ds(LANE * gg, LANE)]

            for j in range(LANE):
                cj = _spl(j)
                r_spl = _lane_bcast(m, cj)
                mk = (r_spl >= plo_spl) & (r_spl < phi_spl)
                oks = lax.reduce_max(mk.astype(jnp.int32), axes=(0,))

                def do(slot, r_spl=r_spl, pv=pv, cj=cj):
                    rl = r_spl - plo_spl
                    for kk in range(4):
                        dvec = _iota16() + LANE * kk
                        if from_tail:
                            vals = plsc.load_gather(tail_v, [rl, dvec])
                        else:
                            vals = plsc.load_gather(chunk_v, [dvec, rl])
                        stag_v[slot, pl.ds(LANE * kk, LANE)] = vals
                    p_spl = _lane_bcast(pv, cj)
                    plsc.store_scatter(ppos_v, [_spl(slot)], p_spl, mask=lane0)
                    return slot + 1

                slot = lax.cond(oks > 0, do, lambda s: s, slot)
            return slot

        return lax.fori_loop(0, ngrp, grp, slot)

    is_last = rng == NR - 1
    slot = 0
    for pp in range(NPASS):
        plo = lo + WCH * pp
        phi = jnp.minimum(plo + WCH, ALIGNED)
        if pp < NPASS - 1:
            pltpu.sync_copy(tab_hbm.at[:, pl.ds(plo, WCH)], chunk_v)
        else:
            @pl.when(jnp.logical_not(is_last))
            def _():
                pltpu.sync_copy(tab_hbm.at[:, pl.ds(plo, WCH)], chunk_v)

            @pl.when(is_last)
            def _():
                pltpu.sync_copy(tab_hbm.at[:, pl.ds(ALIGNED - 128, 128)],
                                chunk_v.at[:, pl.ds(0, 128)])

        slot = gather_pass(plo, phi, slot, False)

    @pl.when(is_last)
    def _():
        pltpu.sync_copy(tail_hbm, tail_v)

    slot = lax.cond(
        is_last,
        lambda s: gather_pass(ALIGNED, NROWS, s, True),
        lambda s: s, slot)

    cps = []
    for b in range(KMAX // LANE):
        cps.append(pltpu.async_copy(
            stag_v.at[pl.ds(LANE * b, LANE)],
            out_hbm.at[ppos_v.at[pl.ds(LANE * b, LANE)]], sem))
    for cp in cps:
        cp.wait()


def _sweep_fn(tabu_hbm, tabm_hbm, tailu_hbm, tailm_hbm, uidx_hbm, midx_hbm,
              uout_hbm, mout_hbm,
              idx_v, cmp_idx_v, cmp_pos_v, ppos_v, chunk_v, tail_v, stag_v,
              sem):
    wid = lax.axis_index("s") * 2 + lax.axis_index("c")
    rng = lax.rem(wid, NR)
    lo = rng * RW
    hi = jnp.minimum(lo + RW, NROWS)
    args = (idx_v, cmp_idx_v, cmp_pos_v, ppos_v, chunk_v, tail_v, stag_v,
            sem, lo, hi, rng)
    _sweep_table(tabu_hbm, tailu_hbm, uidx_hbm, uout_hbm, *args)
    _sweep_table(tabm_hbm, tailm_hbm, midx_hbm, mout_hbm, *args)


def _tail_fn(x_ref, o_ref):
    o_ref[...] = x_ref[...].T[:32, :]


def _tc_tail(tab_t):
    return pl.pallas_call(
        _tail_fn,
        grid=(1,),
        in_specs=[pl.BlockSpec((EMBED, 128), lambda i: (0, NROWS // 128))],
        out_specs=pl.BlockSpec((32, EMBED), lambda i: (0, 0)),
        out_shape=jax.ShapeDtypeStruct((32, EMBED), jnp.float32),
    )(tab_t)


def _sc_sweep_gather(users, movies, tabu_t, tabm_t):
    mesh = plsc.VectorSubcoreMesh(core_axis_name="c", subcore_axis_name="s")
    out = jax.ShapeDtypeStruct((OUTROWS, 2 * EMBED), jnp.float32)
    cp = pltpu.CompilerParams()
    if "needs_layout_passes" in pltpu.CompilerParams.__dataclass_fields__:
        cp = dataclasses.replace(cp, needs_layout_passes=False)
    k = pl.kernel(
        _sweep_fn,
        out_type=(out, out),
        mesh=mesh,
        compiler_params=cp,
        scratch_types=[
            pltpu.VMEM((1024,), jnp.int32),
            pltpu.VMEM((KMAX,), jnp.int32),
            pltpu.VMEM((KMAX,), jnp.int32),
            pltpu.VMEM((KMAX,), jnp.int32),
            pltpu.VMEM((EMBED, WCH), jnp.float32),
            pltpu.VMEM((32, EMBED), jnp.float32),
            pltpu.VMEM((KMAX, 2 * EMBED), jnp.float32),
            pltpu.SemaphoreType.DMA,
        ],
    )
    return k(tabu_t, tabm_t, _tc_tail(tabu_t), _tc_tail(tabm_t), users, movies)


def _bn(x, g, be, eps=1e-5):
    mu = jnp.mean(x, axis=0, keepdims=True)
    var = jnp.mean((x - mu) ** 2, axis=0, keepdims=True)
    return (x - mu) * (g * lax.rsqrt(var + eps)) + be


def _mlp_fn(uo, mo, w1u, w1m, b1, g1, be1, w2, b2, g2, be2,
            w3, b3, g3, be3, wp, bp, o_ref):
    ue = uo[...][:BATCH, :EMBED]
    me = mo[...][:BATCH, :EMBED]
    x = jnp.dot(ue, w1u[...], preferred_element_type=jnp.float32)
    x = x + jnp.dot(me, w1m[...], preferred_element_type=jnp.float32)
    x = jnp.maximum(x + b1[...], 0.0)
    x = _bn(x, g1[...], be1[...])
    x = jnp.dot(x, w2[...], preferred_element_type=jnp.float32)
    x = jnp.maximum(x + b2[...], 0.0)
    x = _bn(x, g2[...], be2[...])
    x = jnp.dot(x, w3[...], preferred_element_type=jnp.float32)
    x = jnp.maximum(x + b3[...], 0.0)
    x = _bn(x, g3[...], be3[...])
    p = jnp.sum(x * wp[...], axis=1, keepdims=True) + bp[...]
    o_ref[...] = jax.nn.sigmoid(p) * 5.0


def _tc_mlp(uo, mo, W1, b1, g1, be1, W2, b2, g2, be2, W3, b3, g3, be3, Wp, bp):
    return pl.pallas_call(
        _mlp_fn,
        out_shape=jax.ShapeDtypeStruct((BATCH, 1), jnp.float32),
    )(uo, mo,
      W1[:EMBED], W1[EMBED:],
      b1.reshape(1, -1), g1.reshape(1, -1), be1.reshape(1, -1),
      W2, b2.reshape(1, -1), g2.reshape(1, -1), be2.reshape(1, -1),
      W3, b3.reshape(1, -1), g3.reshape(1, -1), be3.reshape(1, -1),
      Wp.reshape(1, -1), bp.reshape(1, 1))


def kernel(users, movies, user_table, movie_table,
           W1, b1, g1, be1, W2, b2, g2, be2, W3, b3, g3, be3, Wp, bp):
    u = users.astype(jnp.int32)
    m = movies.astype(jnp.int32)
    uo, mo = _sc_sweep_gather(u, m, user_table.T, movie_table.T)
    return _tc_mlp(uo, mo, W1, b1, g1, be1, W2, b2, g2, be2,
                   W3, b3, g3, be3, Wp, bp)


# speedup vs baseline: 2.7181x; 2.7181x over previous
import jax
import jax.numpy as jnp
from jax import lax
from jax.experimental import pallas as pl
from jax.experimental.pallas import tpu as pltpu
from jax.experimental.pallas import tpu_sc as plsc

BATCH = 4096
EMBED_DIM = 64
PAIR = 2 * EMBED_DIM
HALF = 50176
TBLK = 3584
NBLK = HALF // TBLK
NUM_WORKERS = 32
CHUNK = BATCH // NUM_WORKERS


def _tpose_fn(x1_ref, x2_ref, o_ref):
    o_ref[...] = jnp.concatenate([x1_ref[...].T, x2_ref[...].T], axis=1)


def _tc_pack_transpose(tab_t):
    return pl.pallas_call(
        _tpose_fn,
        grid=(NBLK,),
        in_specs=[
            pl.BlockSpec((EMBED_DIM, TBLK), lambda i: (0, i)),
            pl.BlockSpec((EMBED_DIM, TBLK), lambda i: (0, i + NBLK)),
        ],
        out_specs=pl.BlockSpec((TBLK, PAIR), lambda i: (i, 0)),
        out_shape=jax.ShapeDtypeStruct((HALF, PAIR), jnp.float32),
    )(tab_t, tab_t)


def _sc_gather_fn(tab_hbm, idx_hbm, out_hbm, idx_v, rows_v, sem):
    wid = lax.axis_index("s") * 2 + lax.axis_index("c")
    base = wid * CHUNK
    pltpu.sync_copy(idx_hbm.at[pl.ds(base, CHUNK)], idx_v)
    pltpu.async_copy(tab_hbm.at[idx_v], rows_v, sem).wait()
    pltpu.sync_copy(rows_v, out_hbm.at[pl.ds(base, CHUNK)])


def _sc_gather(pidx, packed):
    mesh = plsc.VectorSubcoreMesh(core_axis_name="c", subcore_axis_name="s")
    k = pl.kernel(
        _sc_gather_fn,
        out_type=jax.ShapeDtypeStruct((BATCH, PAIR), jnp.float32),
        mesh=mesh,
        scratch_types=[
            pltpu.VMEM((CHUNK,), jnp.int32),
            pltpu.VMEM((CHUNK, PAIR), jnp.float32),
            pltpu.SemaphoreType.DMA,
        ],
    )
    return k(packed, pidx)


def _bn(x, g, be, eps=1e-5):
    mu = jnp.mean(x, axis=0, keepdims=True)
    var = jnp.mean((x - mu) ** 2, axis=0, keepdims=True)
    return (x - mu) * (g * lax.rsqrt(var + eps)) + be


def _mlp_fn(pue, pme, par_u, par_m, w1u, w1m, b1, g1, be1, w2, b2, g2, be2,
            w3, b3, g3, be3, wp, bp, o_ref):
    pu = pue[...]
    pm = pme[...]
    ue = jnp.where(par_u[...] > 0, pu[:, EMBED_DIM:], pu[:, :EMBED_DIM])
    me = jnp.where(par_m[...] > 0, pm[:, EMBED_DIM:], pm[:, :EMBED_DIM])
    x = jnp.dot(ue, w1u[...], preferred_element_type=jnp.float32)
    x = x + jnp.dot(me, w1m[...], preferred_element_type=jnp.float32)
    x = jnp.maximum(x + b1[...], 0.0)
    x = _bn(x, g1[...], be1[...])
    x = jnp.dot(x, w2[...], preferred_element_type=jnp.float32)
    x = jnp.maximum(x + b2[...], 0.0)
    x = _bn(x, g2[...], be2[...])
    x = jnp.dot(x, w3[...], preferred_element_type=jnp.float32)
    x = jnp.maximum(x + b3[...], 0.0)
    x = _bn(x, g3[...], be3[...])
    p = jnp.sum(x * wp[...], axis=1, keepdims=True) + bp[...]
    o_ref[...] = jax.nn.sigmoid(p) * 5.0


def _tc_mlp(pue, pme, par_u, par_m,
            W1, b1, g1, be1, W2, b2, g2, be2, W3, b3, g3, be3, Wp, bp):
    return pl.pallas_call(
        _mlp_fn,
        out_shape=jax.ShapeDtypeStruct((BATCH, 1), jnp.float32),
    )(pue, pme, par_u, par_m,
      W1[:EMBED_DIM], W1[EMBED_DIM:],
      b1.reshape(1, -1), g1.reshape(1, -1), be1.reshape(1, -1),
      W2, b2.reshape(1, -1), g2.reshape(1, -1), be2.reshape(1, -1),
      W3, b3.reshape(1, -1), g3.reshape(1, -1), be3.reshape(1, -1),
      Wp.reshape(1, -1), bp.reshape(1, 1))


def kernel(users, movies, user_table, movie_table,
           W1, b1, g1, be1, W2, b2, g2, be2, W3, b3, g3, be3, Wp, bp):
    u = users.astype(jnp.int32)
    m = movies.astype(jnp.int32)
    packed_u = _tc_pack_transpose(user_table.T)
    packed_m = _tc_pack_transpose(movie_table.T)
    pue = _sc_gather(jnp.where(u < HALF, u, u - HALF), packed_u)
    pme = _sc_gather(jnp.where(m < HALF, m, m - HALF), packed_m)
    par_u = (u >= HALF).astype(jnp.int32).reshape(BATCH, 1)
    par_m = (m >= HALF).astype(jnp.int32).reshape(BATCH, 1)
    return _tc_mlp(pue, pme, par_u, par_m, W1, b1, g1, be1,
                   W2, b2, g2, be2, W3, b3, g3, be3, Wp, bp)


# speedup vs baseline: 2.8788x; 1.0591x over previous
import jax
import jax.numpy as jnp
from jax import lax
from jax.experimental import pallas as pl
from jax.experimental.pallas import tpu as pltpu
from jax.experimental.pallas import tpu_sc as plsc

BATCH = 4096
EMBED_DIM = 64
PAIR = 2 * EMBED_DIM
HALF = 50176
TBLK = 7168
NBLK = HALF // TBLK
NUM_WORKERS = 32
CHUNK = BATCH // NUM_WORKERS


def _tpose_fn(x1_ref, x2_ref, o_ref):
    o_ref[...] = jnp.concatenate([x1_ref[...].T, x2_ref[...].T], axis=1)


def _tc_pack_transpose(tab_t):
    return pl.pallas_call(
        _tpose_fn,
        grid=(NBLK,),
        in_specs=[
            pl.BlockSpec((EMBED_DIM, TBLK), lambda i: (0, i)),
            pl.BlockSpec((EMBED_DIM, TBLK), lambda i: (0, i + NBLK)),
        ],
        out_specs=pl.BlockSpec((TBLK, PAIR), lambda i: (i, 0)),
        out_shape=jax.ShapeDtypeStruct((HALF, PAIR), jnp.float32),
    )(tab_t, tab_t)


def _sc_gather_fn(tab_hbm, idx_hbm, out_hbm, idx_v, rows_v, sem):
    wid = lax.axis_index("s") * 2 + lax.axis_index("c")
    base = wid * CHUNK
    pltpu.sync_copy(idx_hbm.at[pl.ds(base, CHUNK)], idx_v)
    pltpu.async_copy(tab_hbm.at[idx_v], rows_v, sem).wait()
    pltpu.sync_copy(rows_v, out_hbm.at[pl.ds(base, CHUNK)])


def _sc_gather(pidx, packed):
    mesh = plsc.VectorSubcoreMesh(core_axis_name="c", subcore_axis_name="s")
    k = pl.kernel(
        _sc_gather_fn,
        out_type=jax.ShapeDtypeStruct((BATCH, PAIR), jnp.float32),
        mesh=mesh,
        scratch_types=[
            pltpu.VMEM((CHUNK,), jnp.int32),
            pltpu.VMEM((CHUNK, PAIR), jnp.float32),
            pltpu.SemaphoreType.DMA,
        ],
    )
    return k(packed, pidx)


def _bn(x, g, be, eps=1e-5):
    mu = jnp.mean(x, axis=0, keepdims=True)
    var = jnp.mean((x - mu) ** 2, axis=0, keepdims=True)
    return (x - mu) * (g * lax.rsqrt(var + eps)) + be


def _mlp_fn(pue, pme, par_u, par_m, w1u, w1m, b1, g1, be1, w2, b2, g2, be2,
            w3, b3, g3, be3, wp, bp, o_ref):
    pu = pue[...]
    pm = pme[...]
    ue = jnp.where(par_u[...] > 0, pu[:, EMBED_DIM:], pu[:, :EMBED_DIM])
    me = jnp.where(par_m[...] > 0, pm[:, EMBED_DIM:], pm[:, :EMBED_DIM])
    x = jnp.dot(ue, w1u[...], preferred_element_type=jnp.float32)
    x = x + jnp.dot(me, w1m[...], preferred_element_type=jnp.float32)
    x = jnp.maximum(x + b1[...], 0.0)
    x = _bn(x, g1[...], be1[...])
    x = jnp.dot(x, w2[...], preferred_element_type=jnp.float32)
    x = jnp.maximum(x + b2[...], 0.0)
    x = _bn(x, g2[...], be2[...])
    x = jnp.dot(x, w3[...], preferred_element_type=jnp.float32)
    x = jnp.maximum(x + b3[...], 0.0)
    x = _bn(x, g3[...], be3[...])
    p = jnp.sum(x * wp[...], axis=1, keepdims=True) + bp[...]
    o_ref[...] = jax.nn.sigmoid(p) * 5.0


def _tc_mlp(pue, pme, par_u, par_m,
            W1, b1, g1, be1, W2, b2, g2, be2, W3, b3, g3, be3, Wp, bp):
    return pl.pallas_call(
        _mlp_fn,
        out_shape=jax.ShapeDtypeStruct((BATCH, 1), jnp.float32),
    )(pue, pme, par_u, par_m,
      W1[:EMBED_DIM], W1[EMBED_DIM:],
      b1.reshape(1, -1), g1.reshape(1, -1), be1.reshape(1, -1),
      W2, b2.reshape(1, -1), g2.reshape(1, -1), be2.reshape(1, -1),
      W3, b3.reshape(1, -1), g3.reshape(1, -1), be3.reshape(1, -1),
      Wp.reshape(1, -1), bp.reshape(1, 1))


def kernel(users, movies, user_table, movie_table,
           W1, b1, g1, be1, W2, b2, g2, be2, W3, b3, g3, be3, Wp, bp):
    u = users.astype(jnp.int32)
    m = movies.astype(jnp.int32)
    packed_u = _tc_pack_transpose(user_table.T)
    packed_m = _tc_pack_transpose(movie_table.T)
    pue = _sc_gather(jnp.where(u < HALF, u, u - HALF), packed_u)
    pme = _sc_gather(jnp.where(m < HALF, m, m - HALF), packed_m)
    par_u = (u >= HALF).astype(jnp.int32).reshape(BATCH, 1)
    par_m = (m >= HALF).astype(jnp.int32).reshape(BATCH, 1)
    return _tc_mlp(pue, pme, par_u, par_m, W1, b1, g1, be1,
                   W2, b2, g2, be2, W3, b3, g3, be3, Wp, bp)


# speedup vs baseline: 2.8930x; 1.0049x over previous
import jax
import jax.numpy as jnp
from jax import lax
from jax.experimental import pallas as pl
from jax.experimental.pallas import tpu as pltpu
from jax.experimental.pallas import tpu_sc as plsc

BATCH = 4096
EMBED_DIM = 64
PAIR = 2 * EMBED_DIM
HALF = 50176
TBLK = 12544
NBLK = HALF // TBLK
NUM_WORKERS = 32
CHUNK = BATCH // NUM_WORKERS


def _tpose_fn(x1_ref, x2_ref, o_ref):
    o_ref[...] = jnp.concatenate([x1_ref[...].T, x2_ref[...].T], axis=1)


def _tc_pack_transpose(tab_t):
    return pl.pallas_call(
        _tpose_fn,
        grid=(NBLK,),
        in_specs=[
            pl.BlockSpec((EMBED_DIM, TBLK), lambda i: (0, i)),
            pl.BlockSpec((EMBED_DIM, TBLK), lambda i: (0, i + NBLK)),
        ],
        out_specs=pl.BlockSpec((TBLK, PAIR), lambda i: (i, 0)),
        out_shape=jax.ShapeDtypeStruct((HALF, PAIR), jnp.float32),
    )(tab_t, tab_t)


def _sc_gather_fn(tab_hbm, idx_hbm, out_hbm, idx_v, rows_v, sem):
    wid = lax.axis_index("s") * 2 + lax.axis_index("c")
    base = wid * CHUNK
    pltpu.sync_copy(idx_hbm.at[pl.ds(base, CHUNK)], idx_v)
    pltpu.async_copy(tab_hbm.at[idx_v], rows_v, sem).wait()
    pltpu.sync_copy(rows_v, out_hbm.at[pl.ds(base, CHUNK)])


def _sc_gather(pidx, packed):
    mesh = plsc.VectorSubcoreMesh(core_axis_name="c", subcore_axis_name="s")
    k = pl.kernel(
        _sc_gather_fn,
        out_type=jax.ShapeDtypeStruct((BATCH, PAIR), jnp.float32),
        mesh=mesh,
        scratch_types=[
            pltpu.VMEM((CHUNK,), jnp.int32),
            pltpu.VMEM((CHUNK, PAIR), jnp.float32),
            pltpu.SemaphoreType.DMA,
        ],
    )
    return k(packed, pidx)


def _bn(x, g, be, eps=1e-5):
    mu = jnp.mean(x, axis=0, keepdims=True)
    var = jnp.mean((x - mu) ** 2, axis=0, keepdims=True)
    return (x - mu) * (g * lax.rsqrt(var + eps)) + be


def _mlp_fn(pue, pme, par_u, par_m, w1u, w1m, b1, g1, be1, w2, b2, g2, be2,
            w3, b3, g3, be3, wp, bp, o_ref):
    pu = pue[...]
    pm = pme[...]
    ue = jnp.where(par_u[...] > 0, pu[:, EMBED_DIM:], pu[:, :EMBED_DIM])
    me = jnp.where(par_m[...] > 0, pm[:, EMBED_DIM:], pm[:, :EMBED_DIM])
    x = jnp.dot(ue, w1u[...], preferred_element_type=jnp.float32)
    x = x + jnp.dot(me, w1m[...], preferred_element_type=jnp.float32)
    x = jnp.maximum(x + b1[...], 0.0)
    x = _bn(x, g1[...], be1[...])
    x = jnp.dot(x, w2[...], preferred_element_type=jnp.float32)
    x = jnp.maximum(x + b2[...], 0.0)
    x = _bn(x, g2[...], be2[...])
    x = jnp.dot(x, w3[...], preferred_element_type=jnp.float32)
    x = jnp.maximum(x + b3[...], 0.0)
    x = _bn(x, g3[...], be3[...])
    p = jnp.sum(x * wp[...], axis=1, keepdims=True) + bp[...]
    o_ref[...] = jax.nn.sigmoid(p) * 5.0


def _tc_mlp(pue, pme, par_u, par_m,
            W1, b1, g1, be1, W2, b2, g2, be2, W3, b3, g3, be3, Wp, bp):
    return pl.pallas_call(
        _mlp_fn,
        out_shape=jax.ShapeDtypeStruct((BATCH, 1), jnp.float32),
    )(pue, pme, par_u, par_m,
      W1[:EMBED_DIM], W1[EMBED_DIM:],
      b1.reshape(1, -1), g1.reshape(1, -1), be1.reshape(1, -1),
      W2, b2.reshape(1, -1), g2.reshape(1, -1), be2.reshape(1, -1),
      W3, b3.reshape(1, -1), g3.reshape(1, -1), be3.reshape(1, -1),
      Wp.reshape(1, -1), bp.reshape(1, 1))


def kernel(users, movies, user_table, movie_table,
           W1, b1, g1, be1, W2, b2, g2, be2, W3, b3, g3, be3, Wp, bp):
    u = users.astype(jnp.int32)
    m = movies.astype(jnp.int32)
    packed_u = _tc_pack_transpose(user_table.T)
    packed_m = _tc_pack_transpose(movie_table.T)
    pue = _sc_gather(jnp.where(u < HALF, u, u - HALF), packed_u)
    pme = _sc_gather(jnp.where(m < HALF, m, m - HALF), packed_m)
    par_u = (u >= HALF).astype(jnp.int32).reshape(BATCH, 1)
    par_m = (m >= HALF).astype(jnp.int32).reshape(BATCH, 1)
    return _tc_mlp(pue, pme, par_u, par_m, W1, b1, g1, be1,
                   W2, b2, g2, be2, W3, b3, g3, be3, Wp, bp)


# speedup vs baseline: 3.0818x; 1.0653x over previous
import jax
import jax.numpy as jnp
from jax import lax
from jax.experimental import pallas as pl
from jax.experimental.pallas import tpu as pltpu
from jax.experimental.pallas import tpu_sc as plsc

BATCH = 4096
EMBED_DIM = 64
PAIR = 2 * EMBED_DIM
HALF = 50176
TBLK = 12544
NBLK = HALF // TBLK
NUM_WORKERS = 32
CHUNK = BATCH // NUM_WORKERS


def _tpose_fn(x1_ref, x2_ref, o_ref):
    o_ref[...] = jnp.concatenate([x1_ref[...].T, x2_ref[...].T], axis=1)


def _tc_pack_transpose(tab_t):
    return pl.pallas_call(
        _tpose_fn,
        grid=(NBLK,),
        compiler_params=pltpu.CompilerParams(
            dimension_semantics=("parallel",)),
        in_specs=[
            pl.BlockSpec((EMBED_DIM, TBLK), lambda i: (0, i)),
            pl.BlockSpec((EMBED_DIM, TBLK), lambda i: (0, i + NBLK)),
        ],
        out_specs=pl.BlockSpec((TBLK, PAIR), lambda i: (i, 0)),
        out_shape=jax.ShapeDtypeStruct((HALF, PAIR), jnp.float32),
    )(tab_t, tab_t)


def _sc_gather_fn(tab_hbm, idx_hbm, out_hbm, idx_v, rows_v, sem):
    wid = lax.axis_index("s") * 2 + lax.axis_index("c")
    base = wid * CHUNK
    pltpu.sync_copy(idx_hbm.at[pl.ds(base, CHUNK)], idx_v)
    half = jnp.full((16,), HALF, jnp.int32)
    for g in range(CHUNK // 16):
        v = idx_v[pl.ds(16 * g, 16)]
        idx_v[pl.ds(16 * g, 16)] = jnp.where(v >= half, v - half, v)
    pltpu.async_copy(tab_hbm.at[idx_v], rows_v, sem).wait()
    pltpu.sync_copy(rows_v, out_hbm.at[pl.ds(base, CHUNK)])


def _sc_gather(pidx, packed):
    mesh = plsc.VectorSubcoreMesh(core_axis_name="c", subcore_axis_name="s")
    k = pl.kernel(
        _sc_gather_fn,
        out_type=jax.ShapeDtypeStruct((BATCH, PAIR), jnp.float32),
        mesh=mesh,
        scratch_types=[
            pltpu.VMEM((CHUNK,), jnp.int32),
            pltpu.VMEM((CHUNK, PAIR), jnp.float32),
            pltpu.SemaphoreType.DMA,
        ],
    )
    return k(packed, pidx)


def _bn(x, g, be, eps=1e-5):
    mu = jnp.mean(x, axis=0, keepdims=True)
    var = jnp.mean((x - mu) ** 2, axis=0, keepdims=True)
    return (x - mu) * (g * lax.rsqrt(var + eps)) + be


def _mlp_fn(pue, pme, u_ref, m_ref, w1u, w1m, b1, g1, be1, w2, b2, g2, be2,
            w3, b3, g3, be3, wp, bp, o_ref):
    pu = pue[...]
    pm = pme[...]
    par_u = (u_ref[...] >= HALF).astype(jnp.int32).reshape(BATCH, 1)
    par_m = (m_ref[...] >= HALF).astype(jnp.int32).reshape(BATCH, 1)
    ue = jnp.where(par_u > 0, pu[:, EMBED_DIM:], pu[:, :EMBED_DIM])
    me = jnp.where(par_m > 0, pm[:, EMBED_DIM:], pm[:, :EMBED_DIM])
    x = jnp.dot(ue, w1u[...], preferred_element_type=jnp.float32)
    x = x + jnp.dot(me, w1m[...], preferred_element_type=jnp.float32)
    x = jnp.maximum(x + b1[...], 0.0)
    x = _bn(x, g1[...], be1[...])
    x = jnp.dot(x, w2[...], preferred_element_type=jnp.float32)
    x = jnp.maximum(x + b2[...], 0.0)
    x = _bn(x, g2[...], be2[...])
    x = jnp.dot(x, w3[...], preferred_element_type=jnp.float32)
    x = jnp.maximum(x + b3[...], 0.0)
    x = _bn(x, g3[...], be3[...])
    p = jnp.sum(x * wp[...], axis=1, keepdims=True) + bp[...]
    o_ref[...] = jax.nn.sigmoid(p) * 5.0


def _tc_mlp(pue, pme, u, m,
            W1, b1, g1, be1, W2, b2, g2, be2, W3, b3, g3, be3, Wp, bp):
    return pl.pallas_call(
        _mlp_fn,
        out_shape=jax.ShapeDtypeStruct((BATCH, 1), jnp.float32),
    )(pue, pme, u, m,
      W1[:EMBED_DIM], W1[EMBED_DIM:],
      b1.reshape(1, -1), g1.reshape(1, -1), be1.reshape(1, -1),
      W2, b2.reshape(1, -1), g2.reshape(1, -1), be2.reshape(1, -1),
      W3, b3.reshape(1, -1), g3.reshape(1, -1), be3.reshape(1, -1),
      Wp.reshape(1, -1), bp.reshape(1, 1))


def kernel(users, movies, user_table, movie_table,
           W1, b1, g1, be1, W2, b2, g2, be2, W3, b3, g3, be3, Wp, bp):
    u = users.astype(jnp.int32)
    m = movies.astype(jnp.int32)
    packed_u = _tc_pack_transpose(user_table.T)
    packed_m = _tc_pack_transpose(movie_table.T)
    pue = _sc_gather(u, packed_u)
    pme = _sc_gather(m, packed_m)
    return _tc_mlp(pue, pme, u, m, W1, b1, g1, be1,
                   W2, b2, g2, be2, W3, b3, g3, be3, Wp, bp)
